# Initial kernel scaffold; baseline (speedup 1.0000x reference)
import jax
import jax.numpy as jnp
from jax.experimental import pallas as pl


def kernel(x, oh, ow, sample_map):
    raise NotImplementedError("write your pallas kernel here")


# speedup vs baseline: 1.0010x; 1.0010x over previous
import jax
import jax.numpy as jnp
from jax.experimental import pallas as pl


def _idx_body(smx_ref, smy_ref, out_ref):
    xi = jnp.clip(jnp.rint(smx_ref[...]), 0.0, 511.0).astype(jnp.int32)
    yi = jnp.clip(jnp.rint(smy_ref[...]), 0.0, 511.0).astype(jnp.int32)
    out_ref[...] = yi * 512 + xi


def kernel(x, oh, ow, sample_map):
    B, C, IH, IW = x.shape
    K = sample_map.shape[2]
    OH, OW = 512, 512
    n = IH * IW * K
    smx = sample_map[..., 0].reshape(n // 128, 128)
    smy = sample_map[..., 1].reshape(n // 128, 128)
    flat_idx = pl.pallas_call(
        _idx_body,
        out_shape=jax.ShapeDtypeStruct((n // 128, 128), jnp.int32),
    )(smx, smy).reshape(-1)
    vals = jnp.repeat(x.reshape(B, C, IH * IW), K, axis=2) * (1.0 / K)
    out = jnp.zeros((B, C, OH * OW), dtype=x.dtype).at[:, :, flat_idx].add(vals)
    return out.reshape(B, C, OH, OW)


# speedup vs baseline: 14.5139x; 14.4987x over previous
import functools

import jax
import jax.numpy as jnp
from jax import lax
from jax.experimental import pallas as pl
from jax.experimental.pallas import tpu as pltpu
from jax.experimental.pallas import tpu_sc as plsc

NC, NS, L = 2, 16, 16
OH = OW = 512
P = 65536
KS = 4
NPLANES = 192
PER_SC = NPLANES // NC
CHUNK = P // NS
NROW = (CHUNK * KS) // 128
STRIPE = (OH * OW) // NS
_RND = 8388608.0


def _make_sc_call():
    mesh = plsc.VectorSubcoreMesh(
        core_axis_name="c", subcore_axis_name="s", num_cores=NC, num_subcores=NS
    )

    @functools.partial(
        pl.kernel,
        out_type=jax.ShapeDtypeStruct((NPLANES, OH * OW), jnp.float32),
        mesh=mesh,
        scratch_types=[
            pltpu.VMEM((CHUNK,), jnp.float32),
            pltpu.VMEM((CHUNK,), jnp.float32),
            pltpu.VMEM((NROW, 128), jnp.int32),
            pltpu.VMEM((CHUNK,), jnp.float32),
            pltpu.VMEM((CHUNK,), jnp.float32),
            pltpu.VMEM((CHUNK,), jnp.float32),
            pltpu.VMEM_SHARED((OH * OW,), jnp.float32),
        ],
    )
    def sc_kernel(x_hbm, smx_hbm, smy_hbm, out_hbm, sbx, sby, idx, xraw, xs, zbuf, acc):
        c = lax.axis_index("c")
        s = lax.axis_index("s")
        pix0 = s * CHUNK
        stripe0 = s * STRIPE

        for k in range(KS):
            pltpu.sync_copy(smx_hbm.at[k, pl.ds(pix0, CHUNK)], sbx)
            pltpu.sync_copy(smy_hbm.at[k, pl.ds(pix0, CHUNK)], sby)

            def idx_body(i, _, k=k):
                vx = sbx[pl.ds(i * L, L)]
                vy = sby[pl.ds(i * L, L)]
                vx = jnp.minimum(jnp.maximum(vx, 0.0), OW - 1.0)
                vy = jnp.minimum(jnp.maximum(vy, 0.0), OH - 1.0)
                vx = (vx + _RND) - _RND
                vy = (vy + _RND) - _RND
                v = (vy * float(OW) + vx).astype(jnp.int32)
                row = k * (NROW // KS) + lax.div(i, 8)
                idx[row, pl.ds(lax.rem(i, 8) * L, L)] = v
                return 0

            lax.fori_loop(0, CHUNK // L, idx_body, 0)

        def z_body(i, _):
            zbuf[pl.ds(i * L, L)] = jnp.zeros((L,), jnp.float32)
            return 0

        lax.fori_loop(0, CHUNK // L, z_body, 0)

        for q in range(STRIPE // CHUNK):
            pltpu.sync_copy(zbuf, acc.at[pl.ds(stripe0 + q * CHUNK, CHUNK)])
        plsc.subcore_barrier()

        def plane_body(p, _):
            bc = c * PER_SC + p
            pltpu.sync_copy(x_hbm.at[bc, pl.ds(pix0, CHUNK)], xraw)

            def s_body(i, _):
                xs[pl.ds(i * L, L)] = xraw[pl.ds(i * L, L)] * (1.0 / KS)
                return 0

            lax.fori_loop(0, CHUNK // L, s_body, 0)

            def sc_body(r, _):
                src = xs.at[pl.ds(lax.rem(r, NROW // KS) * 128, 128)]
                pltpu.sync_copy(src, acc.at[idx.at[r]], add=True)
                return 0

            lax.fori_loop(0, NROW, sc_body, 0)
            plsc.subcore_barrier()

            pltpu.sync_copy(
                acc.at[pl.ds(stripe0, STRIPE)], out_hbm.at[bc, pl.ds(stripe0, STRIPE)]
            )
            for q in range(STRIPE // CHUNK):
                pltpu.sync_copy(zbuf, acc.at[pl.ds(stripe0 + q * CHUNK, CHUNK)])
            plsc.subcore_barrier()
            return 0

        lax.fori_loop(0, PER_SC, plane_body, 0)

    return sc_kernel


_sc_call = _make_sc_call()


def kernel(x, oh, ow, sample_map):
    B, C, IH, IW = x.shape
    x2 = x.reshape(NPLANES, P)
    smk = sample_map.reshape(P, KS, 2).transpose(1, 0, 2)
    smx = smk[..., 0]
    smy = smk[..., 1]
    out = _sc_call(x2, smx, smy)
    return out.reshape(B, C, OH, OW)


# speedup vs baseline: 25.6824x; 1.7695x over previous
import functools

import jax
import jax.numpy as jnp
from jax import lax
from jax.experimental import pallas as pl
from jax.experimental.pallas import tpu as pltpu
from jax.experimental.pallas import tpu_sc as plsc

NC, NS, L = 2, 16, 16
OH = OW = 512
P = 65536
KS = 4
NPLANES = 192
PER_SC = NPLANES // NC
CHUNK = P // NS
NROW = (CHUNK * KS) // 128
STRIPE = (OH * OW) // NS
_RND = 8388608.0


def _make_sc_call():
    mesh = plsc.VectorSubcoreMesh(
        core_axis_name="c", subcore_axis_name="s", num_cores=NC, num_subcores=NS
    )

    @functools.partial(
        pl.kernel,
        out_type=jax.ShapeDtypeStruct((NPLANES, OH * OW), jnp.float32),
        mesh=mesh,
        scratch_types=[
            pltpu.VMEM((CHUNK,), jnp.float32),
            pltpu.VMEM((CHUNK,), jnp.float32),
            pltpu.VMEM((NROW, 128), jnp.int32),
            pltpu.VMEM((2, CHUNK), jnp.float32),
            pltpu.VMEM((2, CHUNK), jnp.float32),
            pltpu.VMEM((CHUNK,), jnp.float32),
            pltpu.VMEM_SHARED((OH * OW,), jnp.float32),
            pltpu.SemaphoreType.DMA,
            pltpu.SemaphoreType.DMA,
        ],
    )
    def sc_kernel(
        x_hbm, smx_hbm, smy_hbm, out_hbm, sbx, sby, idx, xraw, xs, zbuf, acc,
        semsc, semx,
    ):
        c = lax.axis_index("c")
        s = lax.axis_index("s")
        pix0 = s * CHUNK
        stripe0 = s * STRIPE

        for k in range(KS):
            pltpu.sync_copy(smx_hbm.at[k, pl.ds(pix0, CHUNK)], sbx)
            pltpu.sync_copy(smy_hbm.at[k, pl.ds(pix0, CHUNK)], sby)

            def idx_body(i, _, k=k):
                vx = sbx[pl.ds(i * L, L)]
                vy = sby[pl.ds(i * L, L)]
                vx = jnp.minimum(jnp.maximum(vx, 0.0), OW - 1.0)
                vy = jnp.minimum(jnp.maximum(vy, 0.0), OH - 1.0)
                vx = (vx + _RND) - _RND
                vy = (vy + _RND) - _RND
                v = (vy * float(OW) + vx).astype(jnp.int32)
                row = k * (NROW // KS) + lax.div(i, 8)
                idx[row, pl.ds(lax.rem(i, 8) * L, L)] = v
                return 0

            lax.fori_loop(0, CHUNK // L, idx_body, 0)

        def z_body(i, _):
            zbuf[pl.ds(i * L, L)] = jnp.zeros((L,), jnp.float32)
            return 0

        lax.fori_loop(0, CHUNK // L, z_body, 0)

        for q in range(STRIPE // CHUNK):
            pltpu.sync_copy(zbuf, acc.at[pl.ds(stripe0 + q * CHUNK, CHUNK)])
        plsc.subcore_barrier()

        def scale_body(b):
            def s_body(i, _):
                xs[b, pl.ds(i * L, L)] = xraw[b, pl.ds(i * L, L)] * (1.0 / KS)
                return 0

            lax.fori_loop(0, CHUNK // L, s_body, 0)

        pltpu.sync_copy(x_hbm.at[c * PER_SC, pl.ds(pix0, CHUNK)], xraw.at[0])
        scale_body(0)

        def plane_body(pp, _):
            for half in range(2):
                p = pp * 2 + half
                bc = c * PER_SC + p
                descs = []
                for r in range(NROW):
                    src = xs.at[half, pl.ds((r % (NROW // KS)) * 128, 128)]
                    descs.append(
                        pltpu.async_copy(src, acc.at[idx.at[r]], semsc, add=True)
                    )
                nxt = jnp.minimum(bc + 1, NPLANES - 1)
                pltpu.async_copy(
                    x_hbm.at[nxt, pl.ds(pix0, CHUNK)], xraw.at[1 - half], semx
                ).wait()
                scale_body(1 - half)
                for d in descs:
                    d.wait()
                plsc.subcore_barrier()

                pltpu.sync_copy(
                    acc.at[pl.ds(stripe0, STRIPE)],
                    out_hbm.at[bc, pl.ds(stripe0, STRIPE)],
                )
                for q in range(STRIPE // CHUNK):
                    pltpu.sync_copy(zbuf, acc.at[pl.ds(stripe0 + q * CHUNK, CHUNK)])
                plsc.subcore_barrier()
            return 0

        lax.fori_loop(0, PER_SC // 2, plane_body, 0)

    return sc_kernel


_sc_call = _make_sc_call()


def kernel(x, oh, ow, sample_map):
    B, C, IH, IW = x.shape
    x2 = x.reshape(NPLANES, P)
    smk = sample_map.reshape(P, KS, 2).transpose(1, 0, 2)
    smx = smk[..., 0]
    smy = smk[..., 1]
    out = _sc_call(x2, smx, smy)
    return out.reshape(B, C, OH, OW)


# speedup vs baseline: 26.7735x; 1.0425x over previous
import functools

import jax
import jax.numpy as jnp
from jax import lax
from jax.experimental import pallas as pl
from jax.experimental.pallas import tpu as pltpu
from jax.experimental.pallas import tpu_sc as plsc

NC, NS, L = 2, 16, 16
OH = OW = 512
P = 65536
KS = 4
NPLANES = 192
PER_SC = NPLANES // NC
CHUNK = P // NS
NROW = (CHUNK * KS) // 128
STRIPE = (OH * OW) // NS
_RND = 8388608.0


def _make_sc_call():
    mesh = plsc.VectorSubcoreMesh(
        core_axis_name="c", subcore_axis_name="s", num_cores=NC, num_subcores=NS
    )

    @functools.partial(
        pl.kernel,
        out_type=jax.ShapeDtypeStruct((NPLANES, OH * OW), jnp.float32),
        mesh=mesh,
        scratch_types=[
            pltpu.VMEM((NROW, 128), jnp.int32),
            pltpu.VMEM((2, CHUNK), jnp.float32),
            pltpu.VMEM((CHUNK,), jnp.float32),
            pltpu.VMEM((CHUNK,), jnp.float32),
            pltpu.VMEM((CHUNK,), jnp.float32),
            pltpu.VMEM((STRIPE,), jnp.float32),
            pltpu.VMEM_SHARED((OH * OW,), jnp.float32),
            pltpu.VMEM_SHARED((OH * OW,), jnp.float32),
            pltpu.VMEM_SHARED((OH * OW,), jnp.float32),
            pltpu.SemaphoreType.DMA,
            pltpu.SemaphoreType.DMA,
            pltpu.SemaphoreType.DMA,
            pltpu.SemaphoreType.DMA,
            pltpu.SemaphoreType.DMA,
            pltpu.SemaphoreType.DMA,
            pltpu.SemaphoreType.DMA,
        ],
    )
    def sc_kernel(
        x_hbm, smx_hbm, smy_hbm, out_hbm,
        idx, xraw, xs0, xs1, xs2, zbuf, acc0, acc1, acc2,
        semsc0, semsc1, semsc2, semo0, semo1, semo2, semx,
    ):
        xss = [xs0, xs1, xs2]
        accs = [acc0, acc1, acc2]
        semsc = [semsc0, semsc1, semsc2]
        semo = [semo0, semo1, semo2]
        c = lax.axis_index("c")
        s = lax.axis_index("s")
        pix0 = s * CHUNK
        stripe0 = s * STRIPE

        for k in range(KS):
            pltpu.sync_copy(smx_hbm.at[k, pl.ds(pix0, CHUNK)], xraw.at[0])
            pltpu.sync_copy(smy_hbm.at[k, pl.ds(pix0, CHUNK)], xraw.at[1])

            def idx_body(i, _, k=k):
                vx = xraw[0, pl.ds(i * L, L)]
                vy = xraw[1, pl.ds(i * L, L)]
                vx = jnp.minimum(jnp.maximum(vx, 0.0), OW - 1.0)
                vy = jnp.minimum(jnp.maximum(vy, 0.0), OH - 1.0)
                vx = (vx + _RND) - _RND
                vy = (vy + _RND) - _RND
                v = (vy * float(OW) + vx).astype(jnp.int32)
                row = k * (NROW // KS) + lax.div(i, 8)
                idx[row, pl.ds(lax.rem(i, 8) * L, L)] = v
                return 0

            lax.fori_loop(0, CHUNK // L, idx_body, 0)

        def z_body(i, _):
            zbuf[pl.ds(i * L, L)] = jnp.zeros((L,), jnp.float32)
            return 0

        lax.fori_loop(0, STRIPE // L, z_body, 0)
        for a in range(3):
            pltpu.sync_copy(zbuf, accs[a].at[pl.ds(stripe0, STRIPE)])

        def load_scale(p_loc, b):
            bc = c * PER_SC + jnp.minimum(p_loc, PER_SC - 1)
            pltpu.async_copy(
                x_hbm.at[bc, pl.ds(pix0, CHUNK)], xraw.at[0], semx
            ).wait()

            xb = xss[b]

            def s_body(i, _):
                xb[pl.ds(i * L, L)] = xraw[0, pl.ds(i * L, L)] * (1.0 / KS)
                return 0

            lax.fori_loop(0, CHUNK // L, s_body, 0)

        def fire(a):
            def f_body(r, _):
                src = xss[a].at[pl.ds(lax.rem(r, NROW // KS) * 128, 128)]
                pltpu.async_copy(src, accs[a].at[idx.at[r]], semsc[a], add=True)
                return 0

            lax.fori_loop(0, NROW, f_body, 0)

        def drain_scatters(a):
            def w_body(r, _):
                pltpu.make_async_copy(
                    xss[a].at[pl.ds(lax.rem(r, NROW // KS) * 128, 128)],
                    accs[a].at[idx.at[r]],
                    semsc[a],
                ).wait()
                return 0

            lax.fori_loop(0, NROW, w_body, 0)

        def copy_out(p_loc, a):
            bc = c * PER_SC + p_loc
            pltpu.async_copy(
                accs[a].at[pl.ds(stripe0, STRIPE)],
                out_hbm.at[bc, pl.ds(stripe0, STRIPE)],
                semo[a],
            )

        def drain_copy_out(a):
            pltpu.make_async_copy(
                accs[a].at[pl.ds(stripe0, STRIPE)],
                out_hbm.at[0, pl.ds(stripe0, STRIPE)],
                semo[a],
            ).wait()

        def rezero(a):
            pltpu.sync_copy(zbuf, accs[a].at[pl.ds(stripe0, STRIPE)])

        plsc.subcore_barrier()

        load_scale(jnp.int32(0), 0)
        fire(0)
        load_scale(jnp.int32(1), 1)
        drain_scatters(0)
        plsc.subcore_barrier()
        copy_out(jnp.int32(0), 0)
        fire(1)
        load_scale(jnp.int32(2), 2)

        def steady(p, a):
            fire(a)
            drain_copy_out((a + 1) % 3)
            rezero((a + 1) % 3)
            drain_scatters((a + 2) % 3)
            plsc.subcore_barrier()
            copy_out(p - 1, (a + 2) % 3)
            load_scale(p + 1, (a + 1) % 3)

        def tri_body(q, _):
            for j in range(3):
                p = q * 3 + 2 + j
                steady(p, (2 + j) % 3)
            return 0

        lax.fori_loop(0, (PER_SC - 2 - 1) // 3, tri_body, 0)

        steady(jnp.int32(95), 2)

        drain_scatters(95 % 3)
        plsc.subcore_barrier()
        copy_out(jnp.int32(95), 95 % 3)
        drain_copy_out(1)
        drain_copy_out(2)

    return sc_kernel


_sc_call = _make_sc_call()


def kernel(x, oh, ow, sample_map):
    B, C, IH, IW = x.shape
    x2 = x.reshape(NPLANES, P)
    smk = sample_map.reshape(P, KS, 2).transpose(1, 0, 2)
    smx = smk[..., 0]
    smy = smk[..., 1]
    out = _sc_call(x2, smx, smy)
    return out.reshape(B, C, OH, OW)


# speedup vs baseline: 27.4929x; 1.0269x over previous
import functools

import jax
import jax.numpy as jnp
from jax import lax
from jax.experimental import pallas as pl
from jax.experimental.pallas import tpu as pltpu
from jax.experimental.pallas import tpu_sc as plsc

NC, NS, L = 2, 16, 16
OH = OW = 512
P = 65536
KS = 4
NPLANES = 192
PER_SC = NPLANES // NC
CHUNK = P // NS
NROW = (CHUNK * KS) // 128
STRIPE = (OH * OW) // NS
_RND = 8388608.0


def _make_sc_call():
    mesh = plsc.VectorSubcoreMesh(
        core_axis_name="c", subcore_axis_name="s", num_cores=NC, num_subcores=NS
    )

    @functools.partial(
        pl.kernel,
        out_type=jax.ShapeDtypeStruct((NPLANES, OH * OW), jnp.float32),
        mesh=mesh,
        scratch_types=[
            pltpu.VMEM((NROW, 128), jnp.int32),
            pltpu.VMEM((2, CHUNK), jnp.float32),
            pltpu.VMEM((CHUNK,), jnp.float32),
            pltpu.VMEM((CHUNK,), jnp.float32),
            pltpu.VMEM((CHUNK,), jnp.float32),
            pltpu.VMEM((STRIPE,), jnp.float32),
            pltpu.VMEM_SHARED((OH * OW,), jnp.float32),
            pltpu.VMEM_SHARED((OH * OW,), jnp.float32),
            pltpu.VMEM_SHARED((OH * OW,), jnp.float32),
            pltpu.SemaphoreType.DMA,
            pltpu.SemaphoreType.DMA,
            pltpu.SemaphoreType.DMA,
            pltpu.SemaphoreType.DMA,
            pltpu.SemaphoreType.DMA,
            pltpu.SemaphoreType.DMA,
            pltpu.SemaphoreType.DMA,
        ],
    )
    def sc_kernel(
        x_hbm, smx_hbm, smy_hbm, out_hbm,
        idx, xraw, xs0, xs1, xs2, zbuf, acc0, acc1, acc2,
        semsc0, semsc1, semsc2, semo0, semo1, semo2, semx,
    ):
        xss = [xs0, xs1, xs2]
        accs = [acc0, acc1, acc2]
        semsc = [semsc0, semsc1, semsc2]
        semo = [semo0, semo1, semo2]
        c = lax.axis_index("c")
        s = lax.axis_index("s")
        pix0 = s * CHUNK
        stripe0 = s * STRIPE

        for k in range(KS):
            pltpu.sync_copy(smx_hbm.at[k, pl.ds(pix0, CHUNK)], xraw.at[0])
            pltpu.sync_copy(smy_hbm.at[k, pl.ds(pix0, CHUNK)], xraw.at[1])

            def idx_body(i, _, k=k):
                vx = xraw[0, pl.ds(i * L, L)]
                vy = xraw[1, pl.ds(i * L, L)]
                vx = jnp.minimum(jnp.maximum(vx, 0.0), OW - 1.0)
                vy = jnp.minimum(jnp.maximum(vy, 0.0), OH - 1.0)
                vx = (vx + _RND) - _RND
                vy = (vy + _RND) - _RND
                v = (vy * float(OW) + vx).astype(jnp.int32)
                row = k * (NROW // KS) + lax.div(i, 8)
                idx[row, pl.ds(lax.rem(i, 8) * L, L)] = v
                return 0

            lax.fori_loop(0, CHUNK // L, idx_body, 0)

        def z_body(i, _):
            zbuf[pl.ds(i * L, L)] = jnp.zeros((L,), jnp.float32)
            return 0

        lax.fori_loop(0, STRIPE // L, z_body, 0)
        for a in range(3):
            pltpu.sync_copy(zbuf, accs[a].at[pl.ds(stripe0, STRIPE)])

        def load_scale(p_loc, b):
            bc = c * PER_SC + jnp.minimum(p_loc, PER_SC - 1)
            pltpu.async_copy(
                x_hbm.at[bc, pl.ds(pix0, CHUNK)], xraw.at[0], semx
            ).wait()

            xb = xss[b]

            def s_body(i, _):
                xb[pl.ds(i * L, L)] = xraw[0, pl.ds(i * L, L)] * (1.0 / KS)
                return 0

            lax.fori_loop(0, CHUNK // L, s_body, 0)

        def fire(a):
            def f_body(r, _):
                src = xss[a].at[pl.ds(lax.rem(r, NROW // KS) * 128, 128)]
                pltpu.async_copy(src, accs[a].at[idx.at[r]], semsc[a], add=True)
                return 0

            lax.fori_loop(0, NROW, f_body, 0, unroll=4)

        def drain_scatters(a):
            def w_body(r, _):
                pltpu.make_async_copy(
                    xss[a].at[pl.ds(lax.rem(r, NROW // KS) * 128, 128)],
                    accs[a].at[idx.at[r]],
                    semsc[a],
                ).wait()
                return 0

            lax.fori_loop(0, NROW, w_body, 0, unroll=4)

        def copy_out(p_loc, a):
            bc = c * PER_SC + p_loc
            pltpu.async_copy(
                accs[a].at[pl.ds(stripe0, STRIPE)],
                out_hbm.at[bc, pl.ds(stripe0, STRIPE)],
                semo[a],
            )

        def drain_copy_out(a):
            pltpu.make_async_copy(
                accs[a].at[pl.ds(stripe0, STRIPE)],
                out_hbm.at[0, pl.ds(stripe0, STRIPE)],
                semo[a],
            ).wait()

        def rezero(a):
            pltpu.sync_copy(zbuf, accs[a].at[pl.ds(stripe0, STRIPE)])

        plsc.subcore_barrier()

        load_scale(jnp.int32(0), 0)
        fire(0)
        load_scale(jnp.int32(1), 1)
        drain_scatters(0)
        plsc.subcore_barrier()
        copy_out(jnp.int32(0), 0)
        fire(1)
        load_scale(jnp.int32(2), 2)

        def steady(p, a):
            fire(a)
            drain_copy_out((a + 1) % 3)
            rezero((a + 1) % 3)
            drain_scatters((a + 2) % 3)
            plsc.subcore_barrier()
            copy_out(p - 1, (a + 2) % 3)
            load_scale(p + 1, (a + 1) % 3)

        def tri_body(q, _):
            for j in range(3):
                p = q * 3 + 2 + j
                steady(p, (2 + j) % 3)
            return 0

        lax.fori_loop(0, (PER_SC - 2 - 1) // 3, tri_body, 0)

        steady(jnp.int32(95), 2)

        drain_scatters(95 % 3)
        plsc.subcore_barrier()
        copy_out(jnp.int32(95), 95 % 3)
        drain_copy_out(1)
        drain_copy_out(2)

    return sc_kernel


_sc_call = _make_sc_call()


def kernel(x, oh, ow, sample_map):
    B, C, IH, IW = x.shape
    x2 = x.reshape(NPLANES, P)
    smk = sample_map.reshape(P, KS, 2).transpose(1, 0, 2)
    smx = smk[..., 0]
    smy = smk[..., 1]
    out = _sc_call(x2, smx, smy)
    return out.reshape(B, C, OH, OW)
